# baseline (speedup 1.0000x reference)
import functools

import jax
import jax.numpy as jnp
from jax import lax
from jax.experimental import pallas as pl
from jax.experimental.pallas import tpu as pltpu
from jax.experimental.pallas import tpu_sc as plsc

DIM = 32
LANES = 16
NUM_WORKERS = 32
IDX_CHUNK = 128


def _mf_body(users_hbm, items_hbm, ut_hbm, it_hbm,
             ratings_hbm, inter_hbm,
             uidx_v, iidx_v, urows_v, irows_v, inter_v, ratings_v,
             usem, isem):
    b_per_w = uidx_v.shape[0]
    wid = lax.axis_index("s") * 2 + lax.axis_index("c")
    base = wid * b_per_w

    pltpu.sync_copy(users_hbm.at[pl.ds(base, b_per_w)], uidx_v)
    pltpu.sync_copy(items_hbm.at[pl.ds(base, b_per_w)], iidx_v)

    n_chunks = b_per_w // IDX_CHUNK
    copies = []
    for j in range(n_chunks):
        sl = pl.ds(j * IDX_CHUNK, IDX_CHUNK)
        copies.append(
            pltpu.async_copy(ut_hbm.at[uidx_v.at[sl]], urows_v.at[sl], usem))
        copies.append(
            pltpu.async_copy(it_hbm.at[iidx_v.at[sl]], irows_v.at[sl], isem))
    for cp in copies:
        cp.wait()

    lane_iota = lax.iota(jnp.int32, LANES)
    bitrev = (((lane_iota & 1) << 3) | ((lane_iota & 2) << 1)
              | ((lane_iota & 4) >> 1) | ((lane_iota & 8) >> 3))

    def _perm(x, idx):
        return x.at[idx].get(mode="promise_in_bounds")

    def group_body(g, carry):
        row0 = g * LANES
        vecs = []
        for t in range(LANES):
            r = row0 + t
            u0 = urows_v[r, pl.ds(0, LANES)]
            u1 = urows_v[r, pl.ds(LANES, LANES)]
            v0 = irows_v[r, pl.ds(0, LANES)]
            v1 = irows_v[r, pl.ds(LANES, LANES)]
            p0 = u0 * v0
            p1 = u1 * v1
            inter_v[pl.ds(r * DIM, LANES)] = p0
            inter_v[pl.ds(r * DIM + LANES, LANES)] = p1
            vecs.append(p0 + p1)
        d = 8
        while len(vecs) > 1:
            sel = (lane_iota & d) == 0
            nxt = []
            for i in range(0, len(vecs), 2):
                x, y = vecs[i], vecs[i + 1]
                xs = _perm(x, lane_iota ^ d)
                ys = _perm(y, lane_iota ^ d)
                nxt.append(jnp.where(sel, x + xs, y + ys))
            vecs = nxt
            d //= 2
        acc = _perm(vecs[0], bitrev)
        ratings_v[pl.ds(row0, LANES)] = 1.0 / (1.0 + jnp.exp(-acc))
        return carry

    lax.fori_loop(0, b_per_w // LANES, group_body, 0)

    pltpu.sync_copy(inter_v, inter_hbm.at[pl.ds(base * DIM, b_per_w * DIM)])
    pltpu.sync_copy(ratings_v, ratings_hbm.at[pl.ds(base, b_per_w)])


def kernel(users, items, user_table, item_table):
    B = users.shape[0]
    b_per_w = B // NUM_WORKERS
    mesh = plsc.VectorSubcoreMesh(core_axis_name="c", subcore_axis_name="s")
    f = functools.partial(
        pl.kernel,
        mesh=mesh,
        compiler_params=pltpu.CompilerParams(use_tc_tiling_on_sc=False),
        out_type=(jax.ShapeDtypeStruct((B,), jnp.float32),
                  jax.ShapeDtypeStruct((B * DIM,), jnp.float32)),
        scratch_types=[
            pltpu.VMEM((b_per_w,), jnp.int32),
            pltpu.VMEM((b_per_w,), jnp.int32),
            pltpu.VMEM((b_per_w, DIM), jnp.float32),
            pltpu.VMEM((b_per_w, DIM), jnp.float32),
            pltpu.VMEM((b_per_w * DIM,), jnp.float32),
            pltpu.VMEM((b_per_w,), jnp.float32),
            pltpu.SemaphoreType.DMA,
            pltpu.SemaphoreType.DMA,
        ],
    )(_mf_body)
    ratings, inter_flat = f(users, items, user_table, item_table)
    return (ratings, inter_flat.reshape(B, DIM))


# speedup vs baseline: 1.3843x; 1.3843x over previous
import functools

import jax
import jax.numpy as jnp
from jax import lax
from jax.experimental import pallas as pl
from jax.experimental.pallas import tpu as pltpu
from jax.experimental.pallas import tpu_sc as plsc

DIM = 32
LANES = 16
NUM_WORKERS = 32
TILE_ROWS = 8
NBUF = 2


def _mf_body(users_hbm, items_hbm, ut_hbm, it_hbm,
             ratings_hbm, inter_hbm,
             uidx_v, iidx_v, ubuf, ibuf, inter_v, ratings_v,
             usems, isems):
    b_per_w = uidx_v.shape[0]
    n_groups = b_per_w // LANES
    wid = lax.axis_index("s") * 2 + lax.axis_index("c")
    base = wid * b_per_w

    pltpu.sync_copy(users_hbm.at[pl.ds(base, b_per_w)], uidx_v)
    pltpu.sync_copy(items_hbm.at[pl.ds(base, b_per_w)], iidx_v)

    lane_iota = lax.iota(jnp.int32, LANES)
    bitrev = (((lane_iota & 1) << 3) | ((lane_iota & 2) << 1)
              | ((lane_iota & 4) >> 1) | ((lane_iota & 8) >> 3))

    def _perm(x, idx):
        return x.at[idx].get(mode="promise_in_bounds")

    def issue(h, slot):
        uvec = uidx_v[pl.ds(h * LANES, LANES)]
        ivec = iidx_v[pl.ds(h * LANES, LANES)]
        for t in range(LANES):
            ub = pl.multiple_of(uvec[t] & -TILE_ROWS, TILE_ROWS)
            ib = pl.multiple_of(ivec[t] & -TILE_ROWS, TILE_ROWS)
            pltpu.async_copy(ut_hbm.at[pl.ds(ub, TILE_ROWS)],
                             ubuf.at[slot, t], usems[slot])
            pltpu.async_copy(it_hbm.at[pl.ds(ib, TILE_ROWS)],
                             ibuf.at[slot, t], isems[slot])

    def process(g, slot):
        row0 = g * LANES
        uvec = uidx_v[pl.ds(row0, LANES)]
        ivec = iidx_v[pl.ds(row0, LANES)]
        for t in range(LANES):
            pltpu.make_async_copy(ut_hbm.at[pl.ds(0, TILE_ROWS)],
                                  ubuf.at[slot, t], usems[slot]).wait()
            pltpu.make_async_copy(it_hbm.at[pl.ds(0, TILE_ROWS)],
                                  ibuf.at[slot, t], isems[slot]).wait()
        vecs = []
        for t in range(LANES):
            r = row0 + t
            ju = uvec[t] & (TILE_ROWS - 1)
            ji = ivec[t] & (TILE_ROWS - 1)
            u0 = ubuf[slot, t, ju, pl.ds(0, LANES)]
            u1 = ubuf[slot, t, ju, pl.ds(LANES, LANES)]
            v0 = ibuf[slot, t, ji, pl.ds(0, LANES)]
            v1 = ibuf[slot, t, ji, pl.ds(LANES, LANES)]
            p0 = u0 * v0
            p1 = u1 * v1
            inter_v[pl.ds(r * DIM, LANES)] = p0
            inter_v[pl.ds(r * DIM + LANES, LANES)] = p1
            vecs.append(p0 + p1)
        d = 8
        while len(vecs) > 1:
            sel = (lane_iota & d) == 0
            nxt = []
            for i in range(0, len(vecs), 2):
                x, y = vecs[i], vecs[i + 1]
                xs = _perm(x, lane_iota ^ d)
                ys = _perm(y, lane_iota ^ d)
                nxt.append(jnp.where(sel, x + xs, y + ys))
            vecs = nxt
            d //= 2
        acc = _perm(vecs[0], bitrev)
        ratings_v[pl.ds(row0, LANES)] = 1.0 / (1.0 + jnp.exp(-acc))

    for s in range(NBUF):
        issue(s, s)

    def loop_body(k, carry):
        for t in range(NBUF):
            g = k * NBUF + t
            process(g, t)
            issue(g + NBUF, t)
        return carry

    lax.fori_loop(0, n_groups // NBUF - 1, loop_body, 0)

    for t in range(NBUF):
        process(n_groups - NBUF + t, t)

    pltpu.sync_copy(inter_v, inter_hbm.at[pl.ds(base * DIM, b_per_w * DIM)])
    pltpu.sync_copy(ratings_v, ratings_hbm.at[pl.ds(base, b_per_w)])


def kernel(users, items, user_table, item_table):
    B = users.shape[0]
    b_per_w = B // NUM_WORKERS
    mesh = plsc.VectorSubcoreMesh(core_axis_name="c", subcore_axis_name="s")
    f = functools.partial(
        pl.kernel,
        mesh=mesh,
        out_type=(jax.ShapeDtypeStruct((B,), jnp.float32),
                  jax.ShapeDtypeStruct((B * DIM,), jnp.float32)),
        scratch_types=[
            pltpu.VMEM((b_per_w,), jnp.int32),
            pltpu.VMEM((b_per_w,), jnp.int32),
            pltpu.VMEM((NBUF, LANES, TILE_ROWS, DIM), jnp.float32),
            pltpu.VMEM((NBUF, LANES, TILE_ROWS, DIM), jnp.float32),
            pltpu.VMEM((b_per_w * DIM,), jnp.float32),
            pltpu.VMEM((b_per_w,), jnp.float32),
            [pltpu.SemaphoreType.DMA] * NBUF,
            [pltpu.SemaphoreType.DMA] * NBUF,
        ],
    )(_mf_body)
    ratings, inter_flat = f(users, items, user_table, item_table)
    return (ratings, inter_flat.reshape(B, DIM))


# speedup vs baseline: 1.4388x; 1.0393x over previous
import functools

import jax
import jax.numpy as jnp
from jax import lax
from jax.experimental import pallas as pl
from jax.experimental.pallas import tpu as pltpu
from jax.experimental.pallas import tpu_sc as plsc

DIM = 32
LANES = 16
NUM_WORKERS = 32


def _mf_body(users_hbm, items_hbm, ut_hbm, it_hbm,
             ratings_hbm, inter_hbm,
             uidx_v, iidx_v, ubuf, ibuf, inter_v, ratings_v,
             usem, isem):
    b_per_w = uidx_v.shape[0]
    n_groups = b_per_w // LANES
    wid = lax.axis_index("s") * 2 + lax.axis_index("c")
    base = wid * b_per_w

    pltpu.sync_copy(users_hbm.at[pl.ds(base, b_per_w)], uidx_v)
    pltpu.sync_copy(items_hbm.at[pl.ds(base, b_per_w)], iidx_v)

    lane_iota = lax.iota(jnp.int32, LANES)
    bitrev = (((lane_iota & 1) << 3) | ((lane_iota & 2) << 1)
              | ((lane_iota & 4) >> 1) | ((lane_iota & 8) >> 3))

    def _perm(x, idx):
        return x.at[idx].get(mode="promise_in_bounds")

    def group_body(g, carry):
        row0 = g * LANES
        uvec = uidx_v[pl.ds(row0, LANES)]
        ivec = iidx_v[pl.ds(row0, LANES)]
        cps = []
        for t in range(LANES):
            cps.append(pltpu.async_copy(
                ut_hbm.at[pl.ds(uvec[t], 1)], ubuf.at[t], usem))
            cps.append(pltpu.async_copy(
                it_hbm.at[pl.ds(ivec[t], 1)], ibuf.at[t], isem))
        for cp in cps:
            cp.wait()
        vecs = []
        for t in range(LANES):
            r = row0 + t
            u0 = ubuf[t, 0, pl.ds(0, LANES)]
            u1 = ubuf[t, 0, pl.ds(LANES, LANES)]
            v0 = ibuf[t, 0, pl.ds(0, LANES)]
            v1 = ibuf[t, 0, pl.ds(LANES, LANES)]
            p0 = u0 * v0
            p1 = u1 * v1
            inter_v[pl.ds(r * DIM, LANES)] = p0
            inter_v[pl.ds(r * DIM + LANES, LANES)] = p1
            vecs.append(p0 + p1)
        d = 8
        while len(vecs) > 1:
            sel = (lane_iota & d) == 0
            nxt = []
            for i in range(0, len(vecs), 2):
                x, y = vecs[i], vecs[i + 1]
                xs = _perm(x, lane_iota ^ d)
                ys = _perm(y, lane_iota ^ d)
                nxt.append(jnp.where(sel, x + xs, y + ys))
            vecs = nxt
            d //= 2
        acc = _perm(vecs[0], bitrev)
        ratings_v[pl.ds(row0, LANES)] = 1.0 / (1.0 + jnp.exp(-acc))
        return carry

    lax.fori_loop(0, n_groups, group_body, 0)

    pltpu.sync_copy(inter_v, inter_hbm.at[pl.ds(base * DIM, b_per_w * DIM)])
    pltpu.sync_copy(ratings_v, ratings_hbm.at[pl.ds(base, b_per_w)])


def kernel(users, items, user_table, item_table):
    B = users.shape[0]
    b_per_w = B // NUM_WORKERS
    mesh = plsc.VectorSubcoreMesh(core_axis_name="c", subcore_axis_name="s")
    f = functools.partial(
        pl.kernel,
        mesh=mesh,
        compiler_params=pltpu.CompilerParams(use_tc_tiling_on_sc=True),
        out_type=(jax.ShapeDtypeStruct((B,), jnp.float32),
                  jax.ShapeDtypeStruct((B * DIM,), jnp.float32)),
        scratch_types=[
            pltpu.VMEM((b_per_w,), jnp.int32),
            pltpu.VMEM((b_per_w,), jnp.int32),
            pltpu.VMEM((LANES, 1, DIM), jnp.float32),
            pltpu.VMEM((LANES, 1, DIM), jnp.float32),
            pltpu.VMEM((b_per_w * DIM,), jnp.float32),
            pltpu.VMEM((b_per_w,), jnp.float32),
            pltpu.SemaphoreType.DMA,
            pltpu.SemaphoreType.DMA,
        ],
    )(_mf_body)
    ratings, inter_flat = f(users, items, user_table, item_table)
    return (ratings, inter_flat.reshape(B, DIM))
